# Initial kernel scaffold; baseline (speedup 1.0000x reference)
import functools

import jax
import jax.numpy as jnp
from jax import lax
from jax.experimental import pallas as pl
from jax.experimental.pallas import tpu as pltpu
from jax.experimental.pallas import tpu_sc as plsc

N = 10000
E = 320000
D = 128

NC = 2
NS = 16
NW = NC * NS
PER_W = E // NW
CH = 80
NCHUNK = PER_W // CH
RPT = N // NS
ZR = 125
HL = 16

_mesh = plsc.VectorSubcoreMesh(core_axis_name="c", subcore_axis_name="s")


@functools.partial(
    pl.kernel,
    out_type=jax.ShapeDtypeStruct((NC, N, HL), jnp.float32),
    mesh=_mesh,
    scratch_types=[
        pltpu.VMEM((NCHUNK, CH), jnp.int32),
        pltpu.VMEM((CH, HL), jnp.float32),
        pltpu.VMEM((RPT, HL), jnp.float32),
        pltpu.VMEM_SHARED((N, HL), jnp.float32),
    ],
)
def _deg_sc(row_hbm, hist_hbm, idx_v, ones_v, zbuf, hist_sh):
    c = lax.axis_index("c")
    s = lax.axis_index("s")
    w = c * NS + s

    one16 = jnp.full((HL,), 1.0, jnp.float32)
    zero16 = jnp.zeros((HL,), jnp.float32)

    @pl.loop(0, CH)
    def _(i):
        ones_v[i] = one16

    @pl.loop(0, RPT)
    def _(i):
        zbuf[i] = zero16

    pltpu.sync_copy(zbuf, hist_sh.at[pl.ds(s * RPT, RPT)])
    plsc.subcore_barrier()

    pltpu.sync_copy(row_hbm.at[w], idx_v)

    @pl.loop(0, NCHUNK)
    def _(ci):
        pltpu.sync_copy(ones_v, hist_sh.at[idx_v.at[ci]], add=True)

    plsc.subcore_barrier()
    pltpu.sync_copy(hist_sh.at[pl.ds(s * RPT, RPT)],
                    hist_hbm.at[c].at[pl.ds(s * RPT, RPT)])


@functools.partial(
    pl.kernel,
    out_type=jax.ShapeDtypeStruct((NC, N, D), jnp.float32),
    mesh=_mesh,
    scratch_types=[
        pltpu.VMEM((NCHUNK, CH), jnp.int32),
        pltpu.VMEM((NCHUNK, CH), jnp.int32),
        pltpu.VMEM((CH, D), jnp.float32),
        pltpu.VMEM((ZR, D), jnp.float32),
        pltpu.VMEM_SHARED((N, D), jnp.float32),
    ],
)
def _spmm_sc(y_hbm, col_hbm, row_hbm, part_hbm, colv, rowv, buf, zbuf, accum):
    c = lax.axis_index("c")
    s = lax.axis_index("s")
    w = c * NS + s

    zero16 = jnp.zeros((16,), jnp.float32)

    @pl.loop(0, ZR)
    def _(i):
        @pl.loop(0, D // 16)
        def _(j):
            zbuf[i, pl.ds(j * 16, 16)] = zero16

    @pl.loop(0, RPT // ZR)
    def _(k):
        pltpu.sync_copy(zbuf, accum.at[pl.ds(s * RPT + k * ZR, ZR)])

    plsc.subcore_barrier()

    pltpu.sync_copy(col_hbm.at[w], colv)
    pltpu.sync_copy(row_hbm.at[w], rowv)

    @pl.loop(0, NCHUNK)
    def _(ci):
        pltpu.sync_copy(y_hbm.at[colv.at[ci]], buf)
        pltpu.sync_copy(buf, accum.at[rowv.at[ci]], add=True)

    plsc.subcore_barrier()

    @pl.loop(0, RPT // ZR)
    def _(k):
        pltpu.sync_copy(accum.at[pl.ds(s * RPT + k * ZR, ZR)],
                        part_hbm.at[c].at[pl.ds(s * RPT + k * ZR, ZR)])


BLK = 1000


def _stage_a_body(hist_ref, x_ref, y1_ref, dinv_ref):
    deg = 1.0 + hist_ref[0, :, 0:1] + hist_ref[1, :, 0:1]
    dinv = lax.rsqrt(deg)
    y1_ref[...] = x_ref[...] * dinv
    dinv_ref[...] = dinv


_stage_a = pl.pallas_call(
    _stage_a_body,
    grid=(N // BLK,),
    in_specs=[
        pl.BlockSpec((NC, BLK, HL), lambda i: (0, i, 0)),
        pl.BlockSpec((BLK, D), lambda i: (i, 0)),
    ],
    out_specs=[
        pl.BlockSpec((BLK, D), lambda i: (i, 0)),
        pl.BlockSpec((BLK, 1), lambda i: (i, 0)),
    ],
    out_shape=[
        jax.ShapeDtypeStruct((N, D), jnp.float32),
        jax.ShapeDtypeStruct((N, 1), jnp.float32),
    ],
)


def _stage_b_body(y1_ref, p_ref, dinv_ref, w1_ref, b1_ref, y2_ref):
    z = y1_ref[...] + p_ref[0] + p_ref[1]
    u = z * dinv_ref[...]
    h = lax.dot_general(u, w1_ref[...], (((1,), (1,)), ((), ())),
                        preferred_element_type=jnp.float32)
    h = jax.nn.gelu(h + b1_ref[...])
    y2_ref[...] = h * dinv_ref[...]


_stage_b = pl.pallas_call(
    _stage_b_body,
    grid=(N // BLK,),
    in_specs=[
        pl.BlockSpec((BLK, D), lambda i: (i, 0)),
        pl.BlockSpec((NC, BLK, D), lambda i: (0, i, 0)),
        pl.BlockSpec((BLK, 1), lambda i: (i, 0)),
        pl.BlockSpec((D, D), lambda i: (0, 0)),
        pl.BlockSpec((1, D), lambda i: (0, 0)),
    ],
    out_specs=pl.BlockSpec((BLK, D), lambda i: (i, 0)),
    out_shape=jax.ShapeDtypeStruct((N, D), jnp.float32),
)


def _stage_c_body(y2_ref, q_ref, dinv_ref, w2_ref, b2_ref, out_ref):
    z = y2_ref[...] + q_ref[0] + q_ref[1]
    u = z * dinv_ref[...]
    o = lax.dot_general(u, w2_ref[...], (((1,), (1,)), ((), ())),
                        preferred_element_type=jnp.float32)
    out_ref[...] = o + b2_ref[...]


_stage_c = pl.pallas_call(
    _stage_c_body,
    grid=(N // BLK,),
    in_specs=[
        pl.BlockSpec((BLK, D), lambda i: (i, 0)),
        pl.BlockSpec((NC, BLK, D), lambda i: (0, i, 0)),
        pl.BlockSpec((BLK, 1), lambda i: (i, 0)),
        pl.BlockSpec((D, D), lambda i: (0, 0)),
        pl.BlockSpec((1, D), lambda i: (0, 0)),
    ],
    out_specs=pl.BlockSpec((BLK, D), lambda i: (i, 0)),
    out_shape=jax.ShapeDtypeStruct((N, D), jnp.float32),
)


def kernel(X, edge_index, W1, b1, W2, b2):
    row3 = edge_index[0].reshape(NW, NCHUNK, CH)
    col3 = edge_index[1].reshape(NW, NCHUNK, CH)
    b1r = b1.reshape(1, D)
    b2r = b2.reshape(1, D)

    hist = _deg_sc(row3)
    y1, dinv = _stage_a(hist, X)
    p = _spmm_sc(y1, col3, row3)
    y2 = _stage_b(y1, p, dinv, W1, b1r)
    q = _spmm_sc(y2, col3, row3)
    out = _stage_c(y2, q, dinv, W2, b2r)
    return out


# speedup vs baseline: 13.5837x; 13.5837x over previous
import functools

import jax
import jax.numpy as jnp
from jax import lax
from jax.experimental import pallas as pl
from jax.experimental.pallas import tpu as pltpu
from jax.experimental.pallas import tpu_sc as plsc

N = 10000
E = 320000
D = 128
DH = D // 2

NC = 2
NS = 16
NW = NC * NS
PER_W = E // NW
CH = 80
NCHUNK = PER_W // CH
NP8 = 10240
RPT = NP8 // NS
ZR = 128
HL = 16

_mesh = plsc.VectorSubcoreMesh(core_axis_name="c", subcore_axis_name="s")


@functools.partial(
    pl.kernel,
    out_type=jax.ShapeDtypeStruct((NC, NP8, HL), jnp.float32),
    mesh=_mesh,
    scratch_types=[
        pltpu.VMEM((NCHUNK, CH), jnp.int32),
        pltpu.VMEM((CH, HL), jnp.float32),
        pltpu.VMEM((RPT, HL), jnp.float32),
        pltpu.VMEM_SHARED((NP8, HL), jnp.float32),
    ],
    compiler_params=pltpu.CompilerParams(use_tc_tiling_on_sc=False),
)
def _deg_sc(row_hbm, hist_hbm, idx_v, ones_v, zbuf, hist_sh):
    c = lax.axis_index("c")
    s = lax.axis_index("s")
    w = c * NS + s

    one16 = jnp.full((HL,), 1.0, jnp.float32)
    zero16 = jnp.zeros((HL,), jnp.float32)

    @pl.loop(0, CH)
    def _(i):
        ones_v[i] = one16

    @pl.loop(0, RPT)
    def _(i):
        zbuf[i] = zero16

    pltpu.sync_copy(zbuf, hist_sh.at[pl.ds(s * RPT, RPT)])
    plsc.subcore_barrier()

    pltpu.sync_copy(row_hbm.at[w], idx_v)

    @pl.loop(0, NCHUNK)
    def _(ci):
        pltpu.sync_copy(ones_v, hist_sh.at[idx_v.at[ci]], add=True)

    plsc.subcore_barrier()
    pltpu.sync_copy(hist_sh.at[pl.ds(s * RPT, RPT)],
                    hist_hbm.at[c].at[pl.ds(s * RPT, RPT)])


@functools.partial(
    pl.kernel,
    out_type=jax.ShapeDtypeStruct((NC, NP8, DH), jnp.float32),
    mesh=_mesh,
    scratch_types=[
        pltpu.VMEM((NCHUNK, CH), jnp.int32),
        pltpu.VMEM((NCHUNK, CH), jnp.int32),
        pltpu.VMEM((CH, DH), jnp.float32),
        pltpu.VMEM((ZR, DH), jnp.float32),
        pltpu.VMEM_SHARED((NP8, DH), jnp.float32),
    ],
    compiler_params=pltpu.CompilerParams(use_tc_tiling_on_sc=False),
)
def _spmm_sc(y_hbm, col_hbm, row_hbm, part_hbm, colv, rowv, buf, zbuf, accum):
    c = lax.axis_index("c")
    s = lax.axis_index("s")
    w = c * NS + s

    zero16 = jnp.zeros((16,), jnp.float32)

    @pl.loop(0, ZR)
    def _(i):
        @pl.loop(0, DH // 16)
        def _(j):
            zbuf[i, pl.ds(j * 16, 16)] = zero16

    @pl.loop(0, RPT // ZR)
    def _(k):
        pltpu.sync_copy(zbuf, accum.at[pl.ds(s * RPT + k * ZR, ZR)])

    plsc.subcore_barrier()

    pltpu.sync_copy(col_hbm.at[w], colv)
    pltpu.sync_copy(row_hbm.at[w], rowv)

    @pl.loop(0, NCHUNK)
    def _(ci):
        pltpu.sync_copy(y_hbm.at[colv.at[ci]], buf)
        pltpu.sync_copy(buf, accum.at[rowv.at[ci]], add=True)

    plsc.subcore_barrier()

    @pl.loop(0, RPT // ZR)
    def _(k):
        pltpu.sync_copy(accum.at[pl.ds(s * RPT + k * ZR, ZR)],
                        part_hbm.at[c].at[pl.ds(s * RPT + k * ZR, ZR)])


BLK = 1000


def _stage_a_body(hist_ref, x_ref, yl_ref, yr_ref, dinv_ref):
    deg = 1.0 + hist_ref[0, :, 0:1] + hist_ref[1, :, 0:1]
    dinv = lax.rsqrt(deg)
    y = x_ref[...] * dinv
    yl_ref[...] = y[:, :DH]
    yr_ref[...] = y[:, DH:]
    dinv_ref[...] = dinv


_stage_a = pl.pallas_call(
    _stage_a_body,
    grid=(N // BLK,),
    in_specs=[
        pl.BlockSpec((NC, BLK, HL), lambda i: (0, i, 0)),
        pl.BlockSpec((BLK, D), lambda i: (i, 0)),
    ],
    out_specs=[
        pl.BlockSpec((BLK, DH), lambda i: (i, 0)),
        pl.BlockSpec((BLK, DH), lambda i: (i, 0)),
        pl.BlockSpec((BLK, 1), lambda i: (i, 0)),
    ],
    out_shape=[
        jax.ShapeDtypeStruct((N, DH), jnp.float32),
        jax.ShapeDtypeStruct((N, DH), jnp.float32),
        jax.ShapeDtypeStruct((N, 1), jnp.float32),
    ],
)


def _stage_b_body(yl_ref, yr_ref, pl_ref, pr_ref, dinv_ref, w1_ref, b1_ref,
                  y2l_ref, y2r_ref):
    zl = yl_ref[...] + pl_ref[0] + pl_ref[1]
    zr = yr_ref[...] + pr_ref[0] + pr_ref[1]
    u = jnp.concatenate([zl, zr], axis=1) * dinv_ref[...]
    h = lax.dot_general(u, w1_ref[...], (((1,), (1,)), ((), ())),
                        preferred_element_type=jnp.float32)
    h = jax.nn.gelu(h + b1_ref[...]) * dinv_ref[...]
    y2l_ref[...] = h[:, :DH]
    y2r_ref[...] = h[:, DH:]


_stage_b = pl.pallas_call(
    _stage_b_body,
    grid=(N // BLK,),
    in_specs=[
        pl.BlockSpec((BLK, DH), lambda i: (i, 0)),
        pl.BlockSpec((BLK, DH), lambda i: (i, 0)),
        pl.BlockSpec((NC, BLK, DH), lambda i: (0, i, 0)),
        pl.BlockSpec((NC, BLK, DH), lambda i: (0, i, 0)),
        pl.BlockSpec((BLK, 1), lambda i: (i, 0)),
        pl.BlockSpec((D, D), lambda i: (0, 0)),
        pl.BlockSpec((1, D), lambda i: (0, 0)),
    ],
    out_specs=[
        pl.BlockSpec((BLK, DH), lambda i: (i, 0)),
        pl.BlockSpec((BLK, DH), lambda i: (i, 0)),
    ],
    out_shape=[
        jax.ShapeDtypeStruct((N, DH), jnp.float32),
        jax.ShapeDtypeStruct((N, DH), jnp.float32),
    ],
)


def _stage_c_body(yl_ref, yr_ref, ql_ref, qr_ref, dinv_ref, w2_ref, b2_ref,
                  out_ref):
    zl = yl_ref[...] + ql_ref[0] + ql_ref[1]
    zr = yr_ref[...] + qr_ref[0] + qr_ref[1]
    u = jnp.concatenate([zl, zr], axis=1) * dinv_ref[...]
    o = lax.dot_general(u, w2_ref[...], (((1,), (1,)), ((), ())),
                        preferred_element_type=jnp.float32)
    out_ref[...] = o + b2_ref[...]


_stage_c = pl.pallas_call(
    _stage_c_body,
    grid=(N // BLK,),
    in_specs=[
        pl.BlockSpec((BLK, DH), lambda i: (i, 0)),
        pl.BlockSpec((BLK, DH), lambda i: (i, 0)),
        pl.BlockSpec((NC, BLK, DH), lambda i: (0, i, 0)),
        pl.BlockSpec((NC, BLK, DH), lambda i: (0, i, 0)),
        pl.BlockSpec((BLK, 1), lambda i: (i, 0)),
        pl.BlockSpec((D, D), lambda i: (0, 0)),
        pl.BlockSpec((1, D), lambda i: (0, 0)),
    ],
    out_specs=pl.BlockSpec((BLK, D), lambda i: (i, 0)),
    out_shape=jax.ShapeDtypeStruct((N, D), jnp.float32),
)


def kernel(X, edge_index, W1, b1, W2, b2):
    row3 = edge_index[0].reshape(NW, NCHUNK, CH)
    col3 = edge_index[1].reshape(NW, NCHUNK, CH)
    b1r = b1.reshape(1, D)
    b2r = b2.reshape(1, D)

    hist = _deg_sc(row3)
    y1l, y1r, dinv = _stage_a(hist, X)
    p_l = _spmm_sc(y1l, col3, row3)
    p_r = _spmm_sc(y1r, col3, row3)
    y2l, y2r = _stage_b(y1l, y1r, p_l, p_r, dinv, W1, b1r)
    q_l = _spmm_sc(y2l, col3, row3)
    q_r = _spmm_sc(y2r, col3, row3)
    out = _stage_c(y2l, y2r, q_l, q_r, dinv, W2, b2r)
    return out


# speedup vs baseline: 21.0947x; 1.5529x over previous
import functools

import jax
import jax.numpy as jnp
from jax import lax
from jax.experimental import pallas as pl
from jax.experimental.pallas import tpu as pltpu
from jax.experimental.pallas import tpu_sc as plsc

N = 10000
E = 320000
D = 128
DH = D // 2

NC = 2
NS = 16
NW = NC * NS
PER_W = E // NW
CH = 80
NCHUNK = PER_W // CH
NP8 = 10240
RPT = NP8 // NS
ZR = 128
HL = 16

_mesh = plsc.VectorSubcoreMesh(core_axis_name="c", subcore_axis_name="s")


@functools.partial(
    pl.kernel,
    out_type=jax.ShapeDtypeStruct((NC, NP8, HL), jnp.float32),
    mesh=_mesh,
    scratch_types=[
        pltpu.VMEM((NCHUNK, CH), jnp.int32),
        pltpu.VMEM((CH, HL), jnp.float32),
        pltpu.VMEM((RPT, HL), jnp.float32),
        pltpu.VMEM_SHARED((NP8, HL), jnp.float32),
    ],
    compiler_params=pltpu.CompilerParams(use_tc_tiling_on_sc=False),
)
def _deg_sc(row_hbm, hist_hbm, idx_v, ones_v, zbuf, hist_sh):
    c = lax.axis_index("c")
    s = lax.axis_index("s")
    w = c * NS + s

    one16 = jnp.full((HL,), 1.0, jnp.float32)
    zero16 = jnp.zeros((HL,), jnp.float32)

    @pl.loop(0, CH)
    def _(i):
        ones_v[i] = one16

    @pl.loop(0, RPT)
    def _(i):
        zbuf[i] = zero16

    pltpu.sync_copy(zbuf, hist_sh.at[pl.ds(s * RPT, RPT)])
    plsc.subcore_barrier()

    pltpu.sync_copy(row_hbm.at[w], idx_v)

    @pl.loop(0, NCHUNK)
    def _(ci):
        pltpu.sync_copy(ones_v, hist_sh.at[idx_v.at[ci]], add=True)

    plsc.subcore_barrier()
    pltpu.sync_copy(hist_sh.at[pl.ds(s * RPT, RPT)],
                    hist_hbm.at[c].at[pl.ds(s * RPT, RPT)])


@functools.partial(
    pl.kernel,
    out_type=jax.ShapeDtypeStruct((NC, NP8, DH), jnp.float32),
    mesh=_mesh,
    scratch_types=[
        pltpu.VMEM((NCHUNK, CH), jnp.int32),
        pltpu.VMEM((NCHUNK, CH), jnp.int32),
        pltpu.VMEM((CH, DH), jnp.float32),
        pltpu.VMEM((CH, DH), jnp.float32),
        pltpu.VMEM((ZR, DH), jnp.float32),
        pltpu.VMEM_SHARED((NP8, DH), jnp.float32),
        pltpu.SemaphoreType.DMA,
        pltpu.SemaphoreType.DMA,
    ],
    compiler_params=pltpu.CompilerParams(use_tc_tiling_on_sc=False),
)
def _spmm_sc(y_hbm, col_hbm, row_hbm, part_hbm, colv, rowv, buf0, buf1, zbuf,
             accum, gs0, gs1):
    c = lax.axis_index("c")
    s = lax.axis_index("s")
    w = c * NS + s

    zero16 = jnp.zeros((16,), jnp.float32)

    @pl.loop(0, ZR)
    def _(i):
        @pl.loop(0, DH // 16)
        def _(j):
            zbuf[i, pl.ds(j * 16, 16)] = zero16

    @pl.loop(0, RPT // ZR)
    def _(k):
        pltpu.sync_copy(zbuf, accum.at[pl.ds(s * RPT + k * ZR, ZR)])

    plsc.subcore_barrier()

    pltpu.sync_copy(col_hbm.at[w], colv)
    pltpu.sync_copy(row_hbm.at[w], rowv)

    pltpu.async_copy(y_hbm.at[colv.at[0]], buf0, gs0)
    pltpu.async_copy(y_hbm.at[colv.at[1]], buf1, gs1)

    @pl.loop(0, (NCHUNK + 1) // 2)
    def _(k):
        c0 = 2 * k
        pltpu.make_async_copy(y_hbm.at[colv.at[c0]], buf0, gs0).wait()
        pltpu.sync_copy(buf0, accum.at[rowv.at[c0]], add=True)

        @pl.when(c0 + 2 < NCHUNK)
        def _():
            pltpu.async_copy(y_hbm.at[colv.at[c0 + 2]], buf0, gs0)

        @pl.when(c0 + 1 < NCHUNK)
        def _():
            pltpu.make_async_copy(y_hbm.at[colv.at[c0 + 1]], buf1, gs1).wait()
            pltpu.sync_copy(buf1, accum.at[rowv.at[c0 + 1]], add=True)

            @pl.when(c0 + 3 < NCHUNK)
            def _():
                pltpu.async_copy(y_hbm.at[colv.at[c0 + 3]], buf1, gs1)

    plsc.subcore_barrier()

    @pl.loop(0, RPT // ZR)
    def _(k):
        pltpu.sync_copy(accum.at[pl.ds(s * RPT + k * ZR, ZR)],
                        part_hbm.at[c].at[pl.ds(s * RPT + k * ZR, ZR)])


BLK = 1000


def _stage_a_body(hist_ref, x_ref, yl_ref, yr_ref, dinv_ref):
    deg = 1.0 + hist_ref[0, :, 0:1] + hist_ref[1, :, 0:1]
    dinv = lax.rsqrt(deg)
    y = x_ref[...] * dinv
    yl_ref[...] = y[:, :DH]
    yr_ref[...] = y[:, DH:]
    dinv_ref[...] = dinv


_stage_a = pl.pallas_call(
    _stage_a_body,
    grid=(N // BLK,),
    in_specs=[
        pl.BlockSpec((NC, BLK, HL), lambda i: (0, i, 0)),
        pl.BlockSpec((BLK, D), lambda i: (i, 0)),
    ],
    out_specs=[
        pl.BlockSpec((BLK, DH), lambda i: (i, 0)),
        pl.BlockSpec((BLK, DH), lambda i: (i, 0)),
        pl.BlockSpec((BLK, 1), lambda i: (i, 0)),
    ],
    out_shape=[
        jax.ShapeDtypeStruct((N, DH), jnp.float32),
        jax.ShapeDtypeStruct((N, DH), jnp.float32),
        jax.ShapeDtypeStruct((N, 1), jnp.float32),
    ],
)


def _stage_b_body(yl_ref, yr_ref, pl_ref, pr_ref, dinv_ref, w1_ref, b1_ref,
                  y2l_ref, y2r_ref):
    zl = yl_ref[...] + pl_ref[0] + pl_ref[1]
    zr = yr_ref[...] + pr_ref[0] + pr_ref[1]
    u = jnp.concatenate([zl, zr], axis=1) * dinv_ref[...]
    h = lax.dot_general(u, w1_ref[...], (((1,), (1,)), ((), ())),
                        preferred_element_type=jnp.float32)
    h = jax.nn.gelu(h + b1_ref[...]) * dinv_ref[...]
    y2l_ref[...] = h[:, :DH]
    y2r_ref[...] = h[:, DH:]


_stage_b = pl.pallas_call(
    _stage_b_body,
    grid=(N // BLK,),
    in_specs=[
        pl.BlockSpec((BLK, DH), lambda i: (i, 0)),
        pl.BlockSpec((BLK, DH), lambda i: (i, 0)),
        pl.BlockSpec((NC, BLK, DH), lambda i: (0, i, 0)),
        pl.BlockSpec((NC, BLK, DH), lambda i: (0, i, 0)),
        pl.BlockSpec((BLK, 1), lambda i: (i, 0)),
        pl.BlockSpec((D, D), lambda i: (0, 0)),
        pl.BlockSpec((1, D), lambda i: (0, 0)),
    ],
    out_specs=[
        pl.BlockSpec((BLK, DH), lambda i: (i, 0)),
        pl.BlockSpec((BLK, DH), lambda i: (i, 0)),
    ],
    out_shape=[
        jax.ShapeDtypeStruct((N, DH), jnp.float32),
        jax.ShapeDtypeStruct((N, DH), jnp.float32),
    ],
)


def _stage_c_body(yl_ref, yr_ref, ql_ref, qr_ref, dinv_ref, w2_ref, b2_ref,
                  out_ref):
    zl = yl_ref[...] + ql_ref[0] + ql_ref[1]
    zr = yr_ref[...] + qr_ref[0] + qr_ref[1]
    u = jnp.concatenate([zl, zr], axis=1) * dinv_ref[...]
    o = lax.dot_general(u, w2_ref[...], (((1,), (1,)), ((), ())),
                        preferred_element_type=jnp.float32)
    out_ref[...] = o + b2_ref[...]


_stage_c = pl.pallas_call(
    _stage_c_body,
    grid=(N // BLK,),
    in_specs=[
        pl.BlockSpec((BLK, DH), lambda i: (i, 0)),
        pl.BlockSpec((BLK, DH), lambda i: (i, 0)),
        pl.BlockSpec((NC, BLK, DH), lambda i: (0, i, 0)),
        pl.BlockSpec((NC, BLK, DH), lambda i: (0, i, 0)),
        pl.BlockSpec((BLK, 1), lambda i: (i, 0)),
        pl.BlockSpec((D, D), lambda i: (0, 0)),
        pl.BlockSpec((1, D), lambda i: (0, 0)),
    ],
    out_specs=pl.BlockSpec((BLK, D), lambda i: (i, 0)),
    out_shape=jax.ShapeDtypeStruct((N, D), jnp.float32),
)


def kernel(X, edge_index, W1, b1, W2, b2):
    row3 = edge_index[0].reshape(NW, NCHUNK, CH)
    col3 = edge_index[1].reshape(NW, NCHUNK, CH)
    b1r = b1.reshape(1, D)
    b2r = b2.reshape(1, D)

    hist = _deg_sc(row3)
    y1l, y1r, dinv = _stage_a(hist, X)
    p_l = _spmm_sc(y1l, col3, row3)
    p_r = _spmm_sc(y1r, col3, row3)
    y2l, y2r = _stage_b(y1l, y1r, p_l, p_r, dinv, W1, b1r)
    q_l = _spmm_sc(y2l, col3, row3)
    q_r = _spmm_sc(y2r, col3, row3)
    out = _stage_c(y2l, y2r, q_l, q_r, dinv, W2, b2r)
    return out


# speedup vs baseline: 25.0869x; 1.1893x over previous
import functools

import jax
import jax.numpy as jnp
from jax import lax
from jax.experimental import pallas as pl
from jax.experimental.pallas import tpu as pltpu
from jax.experimental.pallas import tpu_sc as plsc

N = 10000
E = 320000
D = 128
DH = D // 2

NC = 2
NS = 16
NW = NC * NS
PER_W = E // NW
CH = 80
NCHUNK = PER_W // CH
NP8 = 10240
RPT = NP8 // NS
ZR = 128
HL = 16

_mesh = plsc.VectorSubcoreMesh(core_axis_name="c", subcore_axis_name="s")


@functools.partial(
    pl.kernel,
    out_type=jax.ShapeDtypeStruct((NC, NP8, HL), jnp.float32),
    mesh=_mesh,
    scratch_types=[
        pltpu.VMEM((NCHUNK, CH), jnp.int32),
        pltpu.VMEM((CH, HL), jnp.float32),
        pltpu.VMEM((RPT, HL), jnp.float32),
        pltpu.VMEM_SHARED((NP8, HL), jnp.float32),
    ],
    compiler_params=pltpu.CompilerParams(use_tc_tiling_on_sc=False),
)
def _deg_sc(row_hbm, hist_hbm, idx_v, ones_v, zbuf, hist_sh):
    c = lax.axis_index("c")
    s = lax.axis_index("s")
    w = c * NS + s

    one16 = jnp.full((HL,), 1.0, jnp.float32)
    zero16 = jnp.zeros((HL,), jnp.float32)

    @pl.loop(0, CH)
    def _(i):
        ones_v[i] = one16

    @pl.loop(0, RPT)
    def _(i):
        zbuf[i] = zero16

    pltpu.sync_copy(zbuf, hist_sh.at[pl.ds(s * RPT, RPT)])
    plsc.subcore_barrier()

    pltpu.sync_copy(row_hbm.at[w], idx_v)

    @pl.loop(0, NCHUNK)
    def _(ci):
        pltpu.sync_copy(ones_v, hist_sh.at[idx_v.at[ci]], add=True)

    plsc.subcore_barrier()
    pltpu.sync_copy(hist_sh.at[pl.ds(s * RPT, RPT)],
                    hist_hbm.at[c].at[pl.ds(s * RPT, RPT)])


@functools.partial(
    pl.kernel,
    out_type=jax.ShapeDtypeStruct((NC, NP8, DH), jnp.float32),
    mesh=_mesh,
    scratch_types=[
        pltpu.VMEM((NCHUNK, CH), jnp.int32),
        pltpu.VMEM((NCHUNK, CH), jnp.int32),
        pltpu.VMEM((CH, DH), jnp.float32),
        pltpu.VMEM((CH, DH), jnp.float32),
        pltpu.VMEM((CH, DH), jnp.float32),
        pltpu.VMEM((CH, DH), jnp.float32),
        pltpu.VMEM((ZR, DH), jnp.float32),
        pltpu.VMEM_SHARED((NP8, DH), jnp.float32),
        pltpu.SemaphoreType.DMA,
        pltpu.SemaphoreType.DMA,
        pltpu.SemaphoreType.DMA,
        pltpu.SemaphoreType.DMA,
        pltpu.SemaphoreType.DMA,
        pltpu.SemaphoreType.DMA,
        pltpu.SemaphoreType.DMA,
        pltpu.SemaphoreType.DMA,
    ],
    compiler_params=pltpu.CompilerParams(use_tc_tiling_on_sc=False),
)
def _spmm_sc(y_hbm, col_hbm, row_hbm, part_hbm, colv, rowv, b0, b1, b2, b3,
             zbuf, accum, g0, g1, g2, g3, s0, s1, s2, s3):
    bufs = (b0, b1, b2, b3)
    gsems = (g0, g1, g2, g3)
    ssems = (s0, s1, s2, s3)
    c = lax.axis_index("c")
    s = lax.axis_index("s")
    w = c * NS + s

    zero16 = jnp.zeros((16,), jnp.float32)

    @pl.loop(0, ZR)
    def _(i):
        @pl.loop(0, DH // 16)
        def _(j):
            zbuf[i, pl.ds(j * 16, 16)] = zero16

    @pl.loop(0, RPT // ZR)
    def _(k):
        pltpu.sync_copy(zbuf, accum.at[pl.ds(s * RPT + k * ZR, ZR)])

    plsc.subcore_barrier()

    pltpu.sync_copy(col_hbm.at[w], colv)
    pltpu.sync_copy(row_hbm.at[w], rowv)

    for b in range(4):
        pltpu.async_copy(y_hbm.at[colv.at[b]], bufs[b], gsems[b])

    @pl.loop(0, (NCHUNK + 3) // 4)
    def _(k):
        c_base = 4 * k
        for b in range(4):
            @pl.when(c_base + b < NCHUNK)
            def _(b=b):
                c = c_base + b
                pltpu.make_async_copy(y_hbm.at[colv.at[c]], bufs[b],
                                      gsems[b]).wait()
                pltpu.async_copy(bufs[b], accum.at[rowv.at[c]], ssems[b],
                                 add=True)
        for b in range(4):
            @pl.when(c_base + b + 4 < NCHUNK)
            def _(b=b):
                c = c_base + b + 4
                pltpu.make_async_copy(bufs[b], accum.at[rowv.at[0]],
                                      ssems[b]).wait()
                pltpu.async_copy(y_hbm.at[colv.at[c]], bufs[b], gsems[b])

    for b in range(4):
        pltpu.make_async_copy(bufs[b], accum.at[rowv.at[0]], ssems[b]).wait()

    plsc.subcore_barrier()

    @pl.loop(0, RPT // ZR)
    def _(k):
        pltpu.sync_copy(accum.at[pl.ds(s * RPT + k * ZR, ZR)],
                        part_hbm.at[c].at[pl.ds(s * RPT + k * ZR, ZR)])


BLK = 1000


def _stage_a_body(hist_ref, x_ref, yl_ref, yr_ref, dinv_ref):
    deg = 1.0 + hist_ref[0, :, 0:1] + hist_ref[1, :, 0:1]
    dinv = lax.rsqrt(deg)
    y = x_ref[...] * dinv
    yl_ref[...] = y[:, :DH]
    yr_ref[...] = y[:, DH:]
    dinv_ref[...] = dinv


_stage_a = pl.pallas_call(
    _stage_a_body,
    grid=(N // BLK,),
    in_specs=[
        pl.BlockSpec((NC, BLK, HL), lambda i: (0, i, 0)),
        pl.BlockSpec((BLK, D), lambda i: (i, 0)),
    ],
    out_specs=[
        pl.BlockSpec((BLK, DH), lambda i: (i, 0)),
        pl.BlockSpec((BLK, DH), lambda i: (i, 0)),
        pl.BlockSpec((BLK, 1), lambda i: (i, 0)),
    ],
    out_shape=[
        jax.ShapeDtypeStruct((N, DH), jnp.float32),
        jax.ShapeDtypeStruct((N, DH), jnp.float32),
        jax.ShapeDtypeStruct((N, 1), jnp.float32),
    ],
)


def _stage_b_body(yl_ref, yr_ref, pl_ref, pr_ref, dinv_ref, w1_ref, b1_ref,
                  y2l_ref, y2r_ref):
    zl = yl_ref[...] + pl_ref[0] + pl_ref[1]
    zr = yr_ref[...] + pr_ref[0] + pr_ref[1]
    u = jnp.concatenate([zl, zr], axis=1) * dinv_ref[...]
    h = lax.dot_general(u, w1_ref[...], (((1,), (1,)), ((), ())),
                        preferred_element_type=jnp.float32)
    h = jax.nn.gelu(h + b1_ref[...]) * dinv_ref[...]
    y2l_ref[...] = h[:, :DH]
    y2r_ref[...] = h[:, DH:]


_stage_b = pl.pallas_call(
    _stage_b_body,
    grid=(N // BLK,),
    in_specs=[
        pl.BlockSpec((BLK, DH), lambda i: (i, 0)),
        pl.BlockSpec((BLK, DH), lambda i: (i, 0)),
        pl.BlockSpec((NC, BLK, DH), lambda i: (0, i, 0)),
        pl.BlockSpec((NC, BLK, DH), lambda i: (0, i, 0)),
        pl.BlockSpec((BLK, 1), lambda i: (i, 0)),
        pl.BlockSpec((D, D), lambda i: (0, 0)),
        pl.BlockSpec((1, D), lambda i: (0, 0)),
    ],
    out_specs=[
        pl.BlockSpec((BLK, DH), lambda i: (i, 0)),
        pl.BlockSpec((BLK, DH), lambda i: (i, 0)),
    ],
    out_shape=[
        jax.ShapeDtypeStruct((N, DH), jnp.float32),
        jax.ShapeDtypeStruct((N, DH), jnp.float32),
    ],
)


def _stage_c_body(yl_ref, yr_ref, ql_ref, qr_ref, dinv_ref, w2_ref, b2_ref,
                  out_ref):
    zl = yl_ref[...] + ql_ref[0] + ql_ref[1]
    zr = yr_ref[...] + qr_ref[0] + qr_ref[1]
    u = jnp.concatenate([zl, zr], axis=1) * dinv_ref[...]
    o = lax.dot_general(u, w2_ref[...], (((1,), (1,)), ((), ())),
                        preferred_element_type=jnp.float32)
    out_ref[...] = o + b2_ref[...]


_stage_c = pl.pallas_call(
    _stage_c_body,
    grid=(N // BLK,),
    in_specs=[
        pl.BlockSpec((BLK, DH), lambda i: (i, 0)),
        pl.BlockSpec((BLK, DH), lambda i: (i, 0)),
        pl.BlockSpec((NC, BLK, DH), lambda i: (0, i, 0)),
        pl.BlockSpec((NC, BLK, DH), lambda i: (0, i, 0)),
        pl.BlockSpec((BLK, 1), lambda i: (i, 0)),
        pl.BlockSpec((D, D), lambda i: (0, 0)),
        pl.BlockSpec((1, D), lambda i: (0, 0)),
    ],
    out_specs=pl.BlockSpec((BLK, D), lambda i: (i, 0)),
    out_shape=jax.ShapeDtypeStruct((N, D), jnp.float32),
)


def kernel(X, edge_index, W1, b1, W2, b2):
    row3 = edge_index[0].reshape(NW, NCHUNK, CH)
    col3 = edge_index[1].reshape(NW, NCHUNK, CH)
    b1r = b1.reshape(1, D)
    b2r = b2.reshape(1, D)

    hist = _deg_sc(row3)
    y1l, y1r, dinv = _stage_a(hist, X)
    p_l = _spmm_sc(y1l, col3, row3)
    p_r = _spmm_sc(y1r, col3, row3)
    y2l, y2r = _stage_b(y1l, y1r, p_l, p_r, dinv, W1, b1r)
    q_l = _spmm_sc(y2l, col3, row3)
    q_r = _spmm_sc(y2r, col3, row3)
    out = _stage_c(y2l, y2r, q_l, q_r, dinv, W2, b2r)
    return out


# speedup vs baseline: 25.6896x; 1.0240x over previous
import functools

import jax
import jax.numpy as jnp
from jax import lax
from jax.experimental import pallas as pl
from jax.experimental.pallas import tpu as pltpu
from jax.experimental.pallas import tpu_sc as plsc

N = 10000
E = 320000
D = 128
DH = D // 2

NC = 2
NS = 16
NW = NC * NS
PER_W = E // NW
CH = 125
NCHUNK = PER_W // CH
NP8 = 10240
RPT = NP8 // NS
ZR = 128
HL = 16

_mesh = plsc.VectorSubcoreMesh(core_axis_name="c", subcore_axis_name="s")


@functools.partial(
    pl.kernel,
    out_type=jax.ShapeDtypeStruct((NC, NP8, HL), jnp.float32),
    mesh=_mesh,
    scratch_types=[
        pltpu.VMEM((NCHUNK, CH), jnp.int32),
        pltpu.VMEM((CH, HL), jnp.float32),
        pltpu.VMEM((RPT, HL), jnp.float32),
        pltpu.VMEM_SHARED((NP8, HL), jnp.float32),
    ],
    compiler_params=pltpu.CompilerParams(use_tc_tiling_on_sc=False),
)
def _deg_sc(row_hbm, hist_hbm, idx_v, ones_v, zbuf, hist_sh):
    c = lax.axis_index("c")
    s = lax.axis_index("s")
    w = c * NS + s

    one16 = jnp.full((HL,), 1.0, jnp.float32)
    zero16 = jnp.zeros((HL,), jnp.float32)

    @pl.loop(0, CH)
    def _(i):
        ones_v[i] = one16

    @pl.loop(0, RPT)
    def _(i):
        zbuf[i] = zero16

    pltpu.sync_copy(zbuf, hist_sh.at[pl.ds(s * RPT, RPT)])
    plsc.subcore_barrier()

    pltpu.sync_copy(row_hbm.at[w], idx_v)

    @pl.loop(0, NCHUNK)
    def _(ci):
        pltpu.sync_copy(ones_v, hist_sh.at[idx_v.at[ci]], add=True)

    plsc.subcore_barrier()
    pltpu.sync_copy(hist_sh.at[pl.ds(s * RPT, RPT)],
                    hist_hbm.at[c].at[pl.ds(s * RPT, RPT)])


@functools.partial(
    pl.kernel,
    out_type=jax.ShapeDtypeStruct((NC, NP8, DH), jnp.float32),
    mesh=_mesh,
    scratch_types=[
        pltpu.VMEM((NCHUNK, CH), jnp.int32),
        pltpu.VMEM((NCHUNK, CH), jnp.int32),
        pltpu.VMEM((CH, DH), jnp.float32),
        pltpu.VMEM((CH, DH), jnp.float32),
        pltpu.VMEM((CH, DH), jnp.float32),
        pltpu.VMEM((CH, DH), jnp.float32),
        pltpu.VMEM((ZR, DH), jnp.float32),
        pltpu.VMEM_SHARED((NP8, DH), jnp.float32),
        pltpu.SemaphoreType.DMA,
        pltpu.SemaphoreType.DMA,
        pltpu.SemaphoreType.DMA,
        pltpu.SemaphoreType.DMA,
        pltpu.SemaphoreType.DMA,
        pltpu.SemaphoreType.DMA,
        pltpu.SemaphoreType.DMA,
        pltpu.SemaphoreType.DMA,
    ],
    compiler_params=pltpu.CompilerParams(use_tc_tiling_on_sc=False),
)
def _spmm_sc(y_hbm, col_hbm, row_hbm, part_hbm, colv, rowv, b0, b1, b2, b3,
             zbuf, accum, g0, g1, g2, g3, s0, s1, s2, s3):
    bufs = (b0, b1, b2, b3)
    gsems = (g0, g1, g2, g3)
    ssems = (s0, s1, s2, s3)
    c = lax.axis_index("c")
    s = lax.axis_index("s")
    w = c * NS + s

    zero16 = jnp.zeros((16,), jnp.float32)

    @pl.loop(0, ZR)
    def _(i):
        @pl.loop(0, DH // 16)
        def _(j):
            zbuf[i, pl.ds(j * 16, 16)] = zero16

    @pl.loop(0, RPT // ZR)
    def _(k):
        pltpu.sync_copy(zbuf, accum.at[pl.ds(s * RPT + k * ZR, ZR)])

    plsc.subcore_barrier()

    pltpu.sync_copy(col_hbm.at[w], colv)
    pltpu.sync_copy(row_hbm.at[w], rowv)

    for b in range(4):
        pltpu.async_copy(y_hbm.at[colv.at[b]], bufs[b], gsems[b])

    @pl.loop(0, (NCHUNK + 3) // 4)
    def _(k):
        c_base = 4 * k
        for b in range(4):
            @pl.when(c_base + b < NCHUNK)
            def _(b=b):
                c = c_base + b
                pltpu.make_async_copy(y_hbm.at[colv.at[c]], bufs[b],
                                      gsems[b]).wait()
                pltpu.async_copy(bufs[b], accum.at[rowv.at[c]], ssems[b],
                                 add=True)
        for b in range(4):
            @pl.when(c_base + b + 4 < NCHUNK)
            def _(b=b):
                c = c_base + b + 4
                pltpu.make_async_copy(bufs[b], accum.at[rowv.at[0]],
                                      ssems[b]).wait()
                pltpu.async_copy(y_hbm.at[colv.at[c]], bufs[b], gsems[b])

    for b in range(4):
        pltpu.make_async_copy(bufs[b], accum.at[rowv.at[0]], ssems[b]).wait()

    plsc.subcore_barrier()

    @pl.loop(0, RPT // ZR)
    def _(k):
        pltpu.sync_copy(accum.at[pl.ds(s * RPT + k * ZR, ZR)],
                        part_hbm.at[c].at[pl.ds(s * RPT + k * ZR, ZR)])


BLK = 1000


def _stage_a_body(hist_ref, x_ref, yl_ref, yr_ref, dinv_ref):
    deg = 1.0 + hist_ref[0, :, 0:1] + hist_ref[1, :, 0:1]
    dinv = lax.rsqrt(deg)
    y = x_ref[...] * dinv
    yl_ref[...] = y[:, :DH]
    yr_ref[...] = y[:, DH:]
    dinv_ref[...] = dinv


_stage_a = pl.pallas_call(
    _stage_a_body,
    grid=(N // BLK,),
    in_specs=[
        pl.BlockSpec((NC, BLK, HL), lambda i: (0, i, 0)),
        pl.BlockSpec((BLK, D), lambda i: (i, 0)),
    ],
    out_specs=[
        pl.BlockSpec((BLK, DH), lambda i: (i, 0)),
        pl.BlockSpec((BLK, DH), lambda i: (i, 0)),
        pl.BlockSpec((BLK, 1), lambda i: (i, 0)),
    ],
    out_shape=[
        jax.ShapeDtypeStruct((N, DH), jnp.float32),
        jax.ShapeDtypeStruct((N, DH), jnp.float32),
        jax.ShapeDtypeStruct((N, 1), jnp.float32),
    ],
)


def _stage_b_body(yl_ref, yr_ref, pl_ref, pr_ref, dinv_ref, w1_ref, b1_ref,
                  y2l_ref, y2r_ref):
    zl = yl_ref[...] + pl_ref[0] + pl_ref[1]
    zr = yr_ref[...] + pr_ref[0] + pr_ref[1]
    u = jnp.concatenate([zl, zr], axis=1) * dinv_ref[...]
    h = lax.dot_general(u, w1_ref[...], (((1,), (1,)), ((), ())),
                        preferred_element_type=jnp.float32)
    h = jax.nn.gelu(h + b1_ref[...]) * dinv_ref[...]
    y2l_ref[...] = h[:, :DH]
    y2r_ref[...] = h[:, DH:]


_stage_b = pl.pallas_call(
    _stage_b_body,
    grid=(N // BLK,),
    in_specs=[
        pl.BlockSpec((BLK, DH), lambda i: (i, 0)),
        pl.BlockSpec((BLK, DH), lambda i: (i, 0)),
        pl.BlockSpec((NC, BLK, DH), lambda i: (0, i, 0)),
        pl.BlockSpec((NC, BLK, DH), lambda i: (0, i, 0)),
        pl.BlockSpec((BLK, 1), lambda i: (i, 0)),
        pl.BlockSpec((D, D), lambda i: (0, 0)),
        pl.BlockSpec((1, D), lambda i: (0, 0)),
    ],
    out_specs=[
        pl.BlockSpec((BLK, DH), lambda i: (i, 0)),
        pl.BlockSpec((BLK, DH), lambda i: (i, 0)),
    ],
    out_shape=[
        jax.ShapeDtypeStruct((N, DH), jnp.float32),
        jax.ShapeDtypeStruct((N, DH), jnp.float32),
    ],
)


def _stage_c_body(yl_ref, yr_ref, ql_ref, qr_ref, dinv_ref, w2_ref, b2_ref,
                  out_ref):
    zl = yl_ref[...] + ql_ref[0] + ql_ref[1]
    zr = yr_ref[...] + qr_ref[0] + qr_ref[1]
    u = jnp.concatenate([zl, zr], axis=1) * dinv_ref[...]
    o = lax.dot_general(u, w2_ref[...], (((1,), (1,)), ((), ())),
                        preferred_element_type=jnp.float32)
    out_ref[...] = o + b2_ref[...]


_stage_c = pl.pallas_call(
    _stage_c_body,
    grid=(N // BLK,),
    in_specs=[
        pl.BlockSpec((BLK, DH), lambda i: (i, 0)),
        pl.BlockSpec((BLK, DH), lambda i: (i, 0)),
        pl.BlockSpec((NC, BLK, DH), lambda i: (0, i, 0)),
        pl.BlockSpec((NC, BLK, DH), lambda i: (0, i, 0)),
        pl.BlockSpec((BLK, 1), lambda i: (i, 0)),
        pl.BlockSpec((D, D), lambda i: (0, 0)),
        pl.BlockSpec((1, D), lambda i: (0, 0)),
    ],
    out_specs=pl.BlockSpec((BLK, D), lambda i: (i, 0)),
    out_shape=jax.ShapeDtypeStruct((N, D), jnp.float32),
)


def kernel(X, edge_index, W1, b1, W2, b2):
    row3 = edge_index[0].reshape(NW, NCHUNK, CH)
    col3 = edge_index[1].reshape(NW, NCHUNK, CH)
    b1r = b1.reshape(1, D)
    b2r = b2.reshape(1, D)

    hist = _deg_sc(row3)
    y1l, y1r, dinv = _stage_a(hist, X)
    p_l = _spmm_sc(y1l, col3, row3)
    p_r = _spmm_sc(y1r, col3, row3)
    y2l, y2r = _stage_b(y1l, y1r, p_l, p_r, dinv, W1, b1r)
    q_l = _spmm_sc(y2l, col3, row3)
    q_r = _spmm_sc(y2r, col3, row3)
    out = _stage_c(y2l, y2r, q_l, q_r, dinv, W2, b2r)
    return out


# speedup vs baseline: 27.8197x; 1.0829x over previous
import functools

import jax
import jax.numpy as jnp
from jax import lax
from jax.experimental import pallas as pl
from jax.experimental.pallas import tpu as pltpu
from jax.experimental.pallas import tpu_sc as plsc

N = 10000
E = 320000
D = 128
DH = D // 2

NC = 2
NS = 16
NW = NC * NS
PER_W = E // NW
CH = 125
NCHUNK = PER_W // CH
PER_S = E // NS
NCHUNK2 = PER_S // CH
NP8 = 10240
RPT = NP8 // NS
ZR = 128
HL = 16

_mesh = plsc.VectorSubcoreMesh(core_axis_name="c", subcore_axis_name="s")


@functools.partial(
    pl.kernel,
    out_type=jax.ShapeDtypeStruct((NC, NP8, HL), jnp.float32),
    mesh=_mesh,
    scratch_types=[
        pltpu.VMEM((NCHUNK, CH), jnp.int32),
        pltpu.VMEM((CH, HL), jnp.float32),
        pltpu.VMEM((RPT, HL), jnp.float32),
        pltpu.VMEM_SHARED((NP8, HL), jnp.float32),
    ],
    compiler_params=pltpu.CompilerParams(use_tc_tiling_on_sc=False),
)
def _deg_sc(row_hbm, hist_hbm, idx_v, ones_v, zbuf, hist_sh):
    c = lax.axis_index("c")
    s = lax.axis_index("s")
    w = c * NS + s

    one16 = jnp.full((HL,), 1.0, jnp.float32)
    zero16 = jnp.zeros((HL,), jnp.float32)

    @pl.loop(0, CH)
    def _(i):
        ones_v[i] = one16

    @pl.loop(0, RPT)
    def _(i):
        zbuf[i] = zero16

    pltpu.sync_copy(zbuf, hist_sh.at[pl.ds(s * RPT, RPT)])
    plsc.subcore_barrier()

    pltpu.sync_copy(row_hbm.at[w], idx_v)

    @pl.loop(0, NCHUNK)
    def _(ci):
        pltpu.sync_copy(ones_v, hist_sh.at[idx_v.at[ci]], add=True)

    plsc.subcore_barrier()
    pltpu.sync_copy(hist_sh.at[pl.ds(s * RPT, RPT)],
                    hist_hbm.at[c].at[pl.ds(s * RPT, RPT)])


@functools.partial(
    pl.kernel,
    out_type=jax.ShapeDtypeStruct((NC, NP8, DH), jnp.float32),
    mesh=_mesh,
    scratch_types=[
        pltpu.VMEM((NCHUNK2, CH), jnp.int32),
        pltpu.VMEM((NCHUNK2, CH), jnp.int32),
        pltpu.VMEM((CH, DH), jnp.float32),
        pltpu.VMEM((CH, DH), jnp.float32),
        pltpu.VMEM((CH, DH), jnp.float32),
        pltpu.VMEM((CH, DH), jnp.float32),
        pltpu.VMEM((ZR, DH), jnp.float32),
        pltpu.VMEM_SHARED((NP8, DH), jnp.float32),
        pltpu.SemaphoreType.DMA,
        pltpu.SemaphoreType.DMA,
        pltpu.SemaphoreType.DMA,
        pltpu.SemaphoreType.DMA,
        pltpu.SemaphoreType.DMA,
        pltpu.SemaphoreType.DMA,
        pltpu.SemaphoreType.DMA,
        pltpu.SemaphoreType.DMA,
    ],
    compiler_params=pltpu.CompilerParams(use_tc_tiling_on_sc=False),
)
def _spmm_sc(yl_hbm, yr_hbm, col_hbm, row_hbm, out_hbm, colv, rowv,
             b0, b1, b2, b3, zbuf, accum, g0, g1, g2, g3, s0, s1, s2, s3):
    bufs = (b0, b1, b2, b3)
    gsems = (g0, g1, g2, g3)
    ssems = (s0, s1, s2, s3)
    c = lax.axis_index("c")
    s = lax.axis_index("s")

    zero16 = jnp.zeros((16,), jnp.float32)

    @pl.loop(0, ZR)
    def _(i):
        @pl.loop(0, DH // 16)
        def _(j):
            zbuf[i, pl.ds(j * 16, 16)] = zero16

    @pl.loop(0, RPT // ZR)
    def _(k):
        pltpu.sync_copy(zbuf, accum.at[pl.ds(s * RPT + k * ZR, ZR)])

    plsc.subcore_barrier()

    pltpu.sync_copy(col_hbm.at[s], colv)
    pltpu.sync_copy(row_hbm.at[s], rowv)

    def run(y_hbm):
        for b in range(4):
            pltpu.async_copy(y_hbm.at[colv.at[b]], bufs[b], gsems[b])

        @pl.loop(0, NCHUNK2 // 4)
        def _(k):
            c_base = 4 * k
            for b in range(4):
                @pl.when(c_base + b < NCHUNK2)
                def _(b=b):
                    ci = c_base + b
                    pltpu.make_async_copy(y_hbm.at[colv.at[ci]], bufs[b],
                                          gsems[b]).wait()
                    pltpu.async_copy(bufs[b], accum.at[rowv.at[ci]], ssems[b],
                                     add=True)
            for b in range(4):
                @pl.when(c_base + b + 4 < NCHUNK2)
                def _(b=b):
                    ci = c_base + b + 4
                    pltpu.make_async_copy(bufs[b], accum.at[rowv.at[0]],
                                          ssems[b]).wait()
                    pltpu.async_copy(y_hbm.at[colv.at[ci]], bufs[b], gsems[b])

        for b in range(4):
            pltpu.make_async_copy(bufs[b], accum.at[rowv.at[0]], ssems[b]).wait()

    @pl.when(c == 0)
    def _():
        run(yl_hbm)

    @pl.when(c == 1)
    def _():
        run(yr_hbm)

    plsc.subcore_barrier()

    @pl.loop(0, RPT // ZR)
    def _(k):
        pltpu.sync_copy(accum.at[pl.ds(s * RPT + k * ZR, ZR)],
                        out_hbm.at[c].at[pl.ds(s * RPT + k * ZR, ZR)])


BLK = 1000


def _stage_a_body(hist_ref, x_ref, yl_ref, yr_ref, dinv_ref):
    deg = 1.0 + hist_ref[0, :, 0:1] + hist_ref[1, :, 0:1]
    dinv = lax.rsqrt(deg)
    y = x_ref[...] * dinv
    yl_ref[...] = y[:, :DH]
    yr_ref[...] = y[:, DH:]
    dinv_ref[...] = dinv


_stage_a = pl.pallas_call(
    _stage_a_body,
    grid=(N // BLK,),
    in_specs=[
        pl.BlockSpec((NC, BLK, HL), lambda i: (0, i, 0)),
        pl.BlockSpec((BLK, D), lambda i: (i, 0)),
    ],
    out_specs=[
        pl.BlockSpec((BLK, DH), lambda i: (i, 0)),
        pl.BlockSpec((BLK, DH), lambda i: (i, 0)),
        pl.BlockSpec((BLK, 1), lambda i: (i, 0)),
    ],
    out_shape=[
        jax.ShapeDtypeStruct((N, DH), jnp.float32),
        jax.ShapeDtypeStruct((N, DH), jnp.float32),
        jax.ShapeDtypeStruct((N, 1), jnp.float32),
    ],
)


def _stage_b_body(yl_ref, yr_ref, p_ref, dinv_ref, w1_ref, b1_ref,
                  y2l_ref, y2r_ref):
    zl = yl_ref[...] + p_ref[0]
    zr = yr_ref[...] + p_ref[1]
    u = jnp.concatenate([zl, zr], axis=1) * dinv_ref[...]
    h = lax.dot_general(u, w1_ref[...], (((1,), (1,)), ((), ())),
                        preferred_element_type=jnp.float32)
    h = jax.nn.gelu(h + b1_ref[...]) * dinv_ref[...]
    y2l_ref[...] = h[:, :DH]
    y2r_ref[...] = h[:, DH:]


_stage_b = pl.pallas_call(
    _stage_b_body,
    grid=(N // BLK,),
    in_specs=[
        pl.BlockSpec((BLK, DH), lambda i: (i, 0)),
        pl.BlockSpec((BLK, DH), lambda i: (i, 0)),
        pl.BlockSpec((NC, BLK, DH), lambda i: (0, i, 0)),
        pl.BlockSpec((BLK, 1), lambda i: (i, 0)),
        pl.BlockSpec((D, D), lambda i: (0, 0)),
        pl.BlockSpec((1, D), lambda i: (0, 0)),
    ],
    out_specs=[
        pl.BlockSpec((BLK, DH), lambda i: (i, 0)),
        pl.BlockSpec((BLK, DH), lambda i: (i, 0)),
    ],
    out_shape=[
        jax.ShapeDtypeStruct((N, DH), jnp.float32),
        jax.ShapeDtypeStruct((N, DH), jnp.float32),
    ],
)


def _stage_c_body(yl_ref, yr_ref, q_ref, dinv_ref, w2_ref, b2_ref,
                  out_ref):
    zl = yl_ref[...] + q_ref[0]
    zr = yr_ref[...] + q_ref[1]
    u = jnp.concatenate([zl, zr], axis=1) * dinv_ref[...]
    o = lax.dot_general(u, w2_ref[...], (((1,), (1,)), ((), ())),
                        preferred_element_type=jnp.float32)
    out_ref[...] = o + b2_ref[...]


_stage_c = pl.pallas_call(
    _stage_c_body,
    grid=(N // BLK,),
    in_specs=[
        pl.BlockSpec((BLK, DH), lambda i: (i, 0)),
        pl.BlockSpec((BLK, DH), lambda i: (i, 0)),
        pl.BlockSpec((NC, BLK, DH), lambda i: (0, i, 0)),
        pl.BlockSpec((BLK, 1), lambda i: (i, 0)),
        pl.BlockSpec((D, D), lambda i: (0, 0)),
        pl.BlockSpec((1, D), lambda i: (0, 0)),
    ],
    out_specs=pl.BlockSpec((BLK, D), lambda i: (i, 0)),
    out_shape=jax.ShapeDtypeStruct((N, D), jnp.float32),
)


def kernel(X, edge_index, W1, b1, W2, b2):
    row3 = edge_index[0].reshape(NW, NCHUNK, CH)
    row3s = edge_index[0].reshape(NS, NCHUNK2, CH)
    col3s = edge_index[1].reshape(NS, NCHUNK2, CH)
    b1r = b1.reshape(1, D)
    b2r = b2.reshape(1, D)

    hist = _deg_sc(row3)
    y1l, y1r, dinv = _stage_a(hist, X)
    p = _spmm_sc(y1l, y1r, col3s, row3s)
    y2l, y2r = _stage_b(y1l, y1r, p, dinv, W1, b1r)
    q = _spmm_sc(y2l, y2r, col3s, row3s)
    out = _stage_c(y2l, y2r, q, dinv, W2, b2r)
    return out


# speedup vs baseline: 28.5951x; 1.0279x over previous
import functools

import jax
import jax.numpy as jnp
from jax import lax
from jax.experimental import pallas as pl
from jax.experimental.pallas import tpu as pltpu
from jax.experimental.pallas import tpu_sc as plsc

N = 10000
E = 320000
D = 128
DH = D // 2

NC = 2
NS = 16
NW = NC * NS
PER_W = E // NW
CH = 125
NCHUNK = PER_W // CH
PER_S = E // NS
NCHUNK2 = PER_S // CH
NP8 = 10240
RPT = NP8 // NS
ZR = 128
HL = 16

_mesh = plsc.VectorSubcoreMesh(core_axis_name="c", subcore_axis_name="s")


@functools.partial(
    pl.kernel,
    out_type=jax.ShapeDtypeStruct((NC, NP8, HL), jnp.float32),
    mesh=_mesh,
    scratch_types=[
        pltpu.VMEM((NCHUNK, CH), jnp.int32),
        pltpu.VMEM((CH, HL), jnp.float32),
        pltpu.VMEM((RPT, HL), jnp.float32),
        pltpu.VMEM_SHARED((NP8, HL), jnp.float32),
        pltpu.SemaphoreType.DMA,
    ],
    compiler_params=pltpu.CompilerParams(use_tc_tiling_on_sc=False),
)
def _deg_sc(row_hbm, hist_hbm, idx_v, ones_v, zbuf, hist_sh, sem):
    c = lax.axis_index("c")
    s = lax.axis_index("s")
    w = c * NS + s

    one16 = jnp.full((HL,), 1.0, jnp.float32)
    zero16 = jnp.zeros((HL,), jnp.float32)

    @pl.loop(0, CH)
    def _(i):
        ones_v[i] = one16

    @pl.loop(0, RPT)
    def _(i):
        zbuf[i] = zero16

    pltpu.sync_copy(zbuf, hist_sh.at[pl.ds(s * RPT, RPT)])
    plsc.subcore_barrier()

    pltpu.sync_copy(row_hbm.at[w], idx_v)

    @pl.loop(0, NCHUNK)
    def _(ci):
        pltpu.async_copy(ones_v, hist_sh.at[idx_v.at[ci]], sem, add=True)

    @pl.loop(0, NCHUNK)
    def _(ci):
        pltpu.make_async_copy(ones_v, hist_sh.at[idx_v.at[0]], sem).wait()

    plsc.subcore_barrier()
    pltpu.sync_copy(hist_sh.at[pl.ds(s * RPT, RPT)],
                    hist_hbm.at[c].at[pl.ds(s * RPT, RPT)])


@functools.partial(
    pl.kernel,
    out_type=jax.ShapeDtypeStruct((NC, NP8, DH), jnp.float32),
    mesh=_mesh,
    scratch_types=[
        pltpu.VMEM((NCHUNK2, CH), jnp.int32),
        pltpu.VMEM((NCHUNK2, CH), jnp.int32),
        pltpu.VMEM((CH, DH), jnp.float32),
        pltpu.VMEM((CH, DH), jnp.float32),
        pltpu.VMEM((CH, DH), jnp.float32),
        pltpu.VMEM((CH, DH), jnp.float32),
        pltpu.VMEM((ZR, DH), jnp.float32),
        pltpu.VMEM_SHARED((NP8, DH), jnp.float32),
        pltpu.SemaphoreType.DMA,
        pltpu.SemaphoreType.DMA,
        pltpu.SemaphoreType.DMA,
        pltpu.SemaphoreType.DMA,
        pltpu.SemaphoreType.DMA,
        pltpu.SemaphoreType.DMA,
        pltpu.SemaphoreType.DMA,
        pltpu.SemaphoreType.DMA,
    ],
    compiler_params=pltpu.CompilerParams(use_tc_tiling_on_sc=False),
)
def _spmm_sc(yl_hbm, yr_hbm, col_hbm, row_hbm, out_hbm, colv, rowv,
             b0, b1, b2, b3, zbuf, accum, g0, g1, g2, g3, s0, s1, s2, s3):
    bufs = (b0, b1, b2, b3)
    gsems = (g0, g1, g2, g3)
    ssems = (s0, s1, s2, s3)
    c = lax.axis_index("c")
    s = lax.axis_index("s")

    zero16 = jnp.zeros((16,), jnp.float32)

    @pl.loop(0, ZR)
    def _(i):
        @pl.loop(0, DH // 16)
        def _(j):
            zbuf[i, pl.ds(j * 16, 16)] = zero16

    @pl.loop(0, RPT // ZR)
    def _(k):
        pltpu.sync_copy(zbuf, accum.at[pl.ds(s * RPT + k * ZR, ZR)])

    plsc.subcore_barrier()

    pltpu.sync_copy(col_hbm.at[s], colv)
    pltpu.sync_copy(row_hbm.at[s], rowv)

    def run(y_hbm):
        for b in range(4):
            pltpu.async_copy(y_hbm.at[colv.at[b]], bufs[b], gsems[b])

        @pl.loop(0, NCHUNK2 // 4)
        def _(k):
            c_base = 4 * k
            for b in range(4):
                @pl.when(c_base + b < NCHUNK2)
                def _(b=b):
                    ci = c_base + b
                    pltpu.make_async_copy(y_hbm.at[colv.at[ci]], bufs[b],
                                          gsems[b]).wait()
                    pltpu.async_copy(bufs[b], accum.at[rowv.at[ci]], ssems[b],
                                     add=True)
            for b in range(4):
                @pl.when(c_base + b + 4 < NCHUNK2)
                def _(b=b):
                    ci = c_base + b + 4
                    pltpu.make_async_copy(bufs[b], accum.at[rowv.at[0]],
                                          ssems[b]).wait()
                    pltpu.async_copy(y_hbm.at[colv.at[ci]], bufs[b], gsems[b])

        for b in range(4):
            pltpu.make_async_copy(bufs[b], accum.at[rowv.at[0]], ssems[b]).wait()

    @pl.when(c == 0)
    def _():
        run(yl_hbm)

    @pl.when(c == 1)
    def _():
        run(yr_hbm)

    plsc.subcore_barrier()

    @pl.loop(0, RPT // ZR)
    def _(k):
        pltpu.sync_copy(accum.at[pl.ds(s * RPT + k * ZR, ZR)],
                        out_hbm.at[c].at[pl.ds(s * RPT + k * ZR, ZR)])


BLK = 2000


def _stage_a_body(hist_ref, x_ref, yl_ref, yr_ref, dinv_ref):
    deg = 1.0 + hist_ref[0, :, 0:1] + hist_ref[1, :, 0:1]
    dinv = lax.rsqrt(deg)
    y = x_ref[...] * dinv
    yl_ref[...] = y[:, :DH]
    yr_ref[...] = y[:, DH:]
    dinv_ref[...] = dinv


_stage_a = pl.pallas_call(
    _stage_a_body,
    grid=(N // BLK,),
    in_specs=[
        pl.BlockSpec((NC, BLK, HL), lambda i: (0, i, 0)),
        pl.BlockSpec((BLK, D), lambda i: (i, 0)),
    ],
    out_specs=[
        pl.BlockSpec((BLK, DH), lambda i: (i, 0)),
        pl.BlockSpec((BLK, DH), lambda i: (i, 0)),
        pl.BlockSpec((BLK, 1), lambda i: (i, 0)),
    ],
    out_shape=[
        jax.ShapeDtypeStruct((N, DH), jnp.float32),
        jax.ShapeDtypeStruct((N, DH), jnp.float32),
        jax.ShapeDtypeStruct((N, 1), jnp.float32),
    ],
)


def _stage_b_body(yl_ref, yr_ref, p_ref, dinv_ref, w1_ref, b1_ref,
                  y2l_ref, y2r_ref):
    zl = yl_ref[...] + p_ref[0]
    zr = yr_ref[...] + p_ref[1]
    u = jnp.concatenate([zl, zr], axis=1) * dinv_ref[...]
    h = lax.dot_general(u, w1_ref[...], (((1,), (1,)), ((), ())),
                        preferred_element_type=jnp.float32)
    h = jax.nn.gelu(h + b1_ref[...]) * dinv_ref[...]
    y2l_ref[...] = h[:, :DH]
    y2r_ref[...] = h[:, DH:]


_stage_b = pl.pallas_call(
    _stage_b_body,
    grid=(N // BLK,),
    in_specs=[
        pl.BlockSpec((BLK, DH), lambda i: (i, 0)),
        pl.BlockSpec((BLK, DH), lambda i: (i, 0)),
        pl.BlockSpec((NC, BLK, DH), lambda i: (0, i, 0)),
        pl.BlockSpec((BLK, 1), lambda i: (i, 0)),
        pl.BlockSpec((D, D), lambda i: (0, 0)),
        pl.BlockSpec((1, D), lambda i: (0, 0)),
    ],
    out_specs=[
        pl.BlockSpec((BLK, DH), lambda i: (i, 0)),
        pl.BlockSpec((BLK, DH), lambda i: (i, 0)),
    ],
    out_shape=[
        jax.ShapeDtypeStruct((N, DH), jnp.float32),
        jax.ShapeDtypeStruct((N, DH), jnp.float32),
    ],
)


def _stage_c_body(yl_ref, yr_ref, q_ref, dinv_ref, w2_ref, b2_ref,
                  out_ref):
    zl = yl_ref[...] + q_ref[0]
    zr = yr_ref[...] + q_ref[1]
    u = jnp.concatenate([zl, zr], axis=1) * dinv_ref[...]
    o = lax.dot_general(u, w2_ref[...], (((1,), (1,)), ((), ())),
                        preferred_element_type=jnp.float32)
    out_ref[...] = o + b2_ref[...]


_stage_c = pl.pallas_call(
    _stage_c_body,
    grid=(N // BLK,),
    in_specs=[
        pl.BlockSpec((BLK, DH), lambda i: (i, 0)),
        pl.BlockSpec((BLK, DH), lambda i: (i, 0)),
        pl.BlockSpec((NC, BLK, DH), lambda i: (0, i, 0)),
        pl.BlockSpec((BLK, 1), lambda i: (i, 0)),
        pl.BlockSpec((D, D), lambda i: (0, 0)),
        pl.BlockSpec((1, D), lambda i: (0, 0)),
    ],
    out_specs=pl.BlockSpec((BLK, D), lambda i: (i, 0)),
    out_shape=jax.ShapeDtypeStruct((N, D), jnp.float32),
)


def kernel(X, edge_index, W1, b1, W2, b2):
    row3 = edge_index[0].reshape(NW, NCHUNK, CH)
    row3s = edge_index[0].reshape(NS, NCHUNK2, CH)
    col3s = edge_index[1].reshape(NS, NCHUNK2, CH)
    b1r = b1.reshape(1, D)
    b2r = b2.reshape(1, D)

    hist = _deg_sc(row3)
    y1l, y1r, dinv = _stage_a(hist, X)
    p = _spmm_sc(y1l, y1r, col3s, row3s)
    y2l, y2r = _stage_b(y1l, y1r, p, dinv, W1, b1r)
    q = _spmm_sc(y2l, y2r, col3s, row3s)
    out = _stage_c(y2l, y2r, q, dinv, W2, b2r)
    return out
